# baseline (speedup 1.0000x reference)
import functools

import jax
import jax.numpy as jnp
from jax.experimental import pallas as pl
from jax.experimental.pallas import tpu as pltpu

T, B, D, H, MC, ES = 32, 64, 256, 256, 1024, 256


def _step_kernel(x_ref, g_ref, wx_ref, whh_ref, bih_ref, bhh_ref,
                 wh_ref, wum_ref, fc1w_ref, fc1b_ref,
                 fc2a_ref, fc2bias_ref,
                 out_ref, h_scr, lu_scr):
    t = pl.program_id(0)

    @pl.when(t == 0)
    def _init():
        h_scr[...] = jnp.zeros((B, H), jnp.float32)
        lu_scr[...] = jnp.full((B, MC), -99999.0, jnp.float32)

    h = h_scr[...]
    lu = lu_scr[...]
    xp = jnp.dot(x_ref[0], wx_ref[...])
    hp = jnp.dot(h, wh_ref[...])

    last_use = jax.nn.sigmoid(lu)
    pre = jnp.tanh(xp[:, :ES]
                   + hp[:, :ES]
                   + jnp.dot(last_use, wum_ref[...]))
    read_head = jnp.dot(pre, fc1w_ref[...]) + fc1b_ref[...]
    u = g_ref[0]
    g = -jnp.log(1e-20 - jnp.log(1e-20 + u))
    logits = read_head + g

    m = jnp.max(logits, axis=1, keepdims=True)
    col = jax.lax.broadcasted_iota(jnp.int32, (B, MC), 1)
    pos = jnp.min(jnp.where(logits == m, col, MC), axis=1,
                  keepdims=True)

    idx = jnp.clip(pos - 1, 0, T - 1)
    nodes = [out_ref[s] for s in range(T)]
    for level in range(5):
        take_hi = ((idx >> level) & 1) == 1
        nodes = [jnp.where(take_hi, nodes[2 * i + 1], nodes[2 * i])
                 for i in range(len(nodes) // 2)]
    valid = (pos >= 1) & (pos <= t)
    entry = jnp.where(valid, nodes[0], 0.0)

    lu_scr[...] = jnp.where(col == pos, -1.0, lu - 1.0)

    h_new = (jnp.dot(entry, fc2a_ref[...]) + hp[:, ES:]
             + fc2bias_ref[...])

    wi = xp[:, ES:] + bih_ref[...]
    wh = jnp.dot(h_new, whh_ref[...]) + bhh_ref[...]
    r = jax.nn.sigmoid(wi[:, :H] + wh[:, :H])
    z = jax.nn.sigmoid(wi[:, H:2 * H] + wh[:, H:2 * H])
    n = jnp.tanh(wi[:, 2 * H:] + r * wh[:, 2 * H:])
    h_out = (1.0 - z) * n + z * h_new

    h_scr[...] = h_out
    out_ref[t] = h_out


@functools.partial(jax.jit, static_argnames=())
def kernel(input_, gumbel_u, weight_ih, weight_hh, bias, weight_im,
           weight_hm, weight_um, fc1_w, fc1_b, fc2_w, fc2_b):
    bias_ih = bias[: 3 * H].reshape(1, 3 * H)
    bias_hh = bias[3 * H:].reshape(1, 3 * H)
    fc1b = fc1_b.reshape(1, MC)
    fc2bias = fc2_b.reshape(1, H)
    fc2a = fc2_w[:ES]
    w_x = jnp.concatenate([weight_im, weight_ih], axis=1)
    w_h = jnp.concatenate([weight_hm, fc2_w[ES:]], axis=1)

    full = lambda shape: pl.BlockSpec(shape, lambda t: (0,) * len(shape))
    return pl.pallas_call(
        _step_kernel,
        grid=(T,),
        in_specs=[
            pl.BlockSpec((1, B, D), lambda t: (t, 0, 0)),
            pl.BlockSpec((1, B, MC), lambda t: (t, 0, 0)),
            full((D, ES + 3 * H)),
            full((H, 3 * H)),
            full((1, 3 * H)),
            full((1, 3 * H)),
            full((H, ES + H)),
            full((MC, ES)),
            full((ES, MC)),
            full((1, MC)),
            full((ES, H)),
            full((1, H)),
        ],
        out_specs=pl.BlockSpec((T, B, H), lambda t: (0, 0, 0)),
        out_shape=jax.ShapeDtypeStruct((T, B, H), jnp.float32),
        scratch_shapes=[
            pltpu.VMEM((B, H), jnp.float32),
            pltpu.VMEM((B, MC), jnp.float32),
        ],
        compiler_params=pltpu.CompilerParams(
            dimension_semantics=("arbitrary",),
        ),
    )(input_, gumbel_u, w_x, weight_hh, bias_ih, bias_hh,
      w_h, weight_um, fc1_w, fc1b, fc2a, fc2bias)


# speedup vs baseline: 1.0838x; 1.0838x over previous
import functools

import jax
import jax.numpy as jnp
from jax.experimental import pallas as pl
from jax.experimental.pallas import tpu as pltpu

T, B, D, H, MC, ES = 32, 64, 256, 256, 1024, 256


def _step_kernel(x_cur_ref, x_nxt_ref, g_ref, wx_ref, whh_ref, bih_ref,
                 bhh_ref, wh_ref, wum_ref, fc1w_ref, fc1b_ref,
                 fc2a_ref, fc2bias_ref,
                 out_ref, h_scr, lu_scr, xp_buf, c_buf):
    t = pl.program_id(0)
    par = jax.lax.rem(t, 2)
    nxt = jax.lax.rem(t + 1, 2)

    @pl.when(t == 0)
    def _init():
        h_scr[...] = jnp.zeros((B, H), jnp.float32)
        lu_scr[...] = jnp.full((B, MC), -99999.0, jnp.float32)
        c_buf[0] = jnp.zeros((B, ES), jnp.float32)
        xp_buf[0] = jnp.dot(x_cur_ref[0], wx_ref[...])

    h = h_scr[...]
    lu = lu_scr[...]
    xp = xp_buf[par]
    c = c_buf[par]

    hp = jnp.dot(h, wh_ref[...])
    pre = jnp.tanh(xp[:, :ES] + hp[:, :ES] + c)
    read_head = jnp.dot(pre, fc1w_ref[...]) + fc1b_ref[...]
    u = g_ref[0]
    g = -jnp.log(1e-20 - jnp.log(1e-20 + u))
    logits = read_head + g

    m = jnp.max(logits, axis=1, keepdims=True)
    col = jax.lax.broadcasted_iota(jnp.int32, (B, MC), 1)
    pos = jnp.min(jnp.where(logits == m, col, MC), axis=1,
                  keepdims=True)

    lu_new = jnp.where(col == pos, -1.0, lu - 1.0)
    lu_scr[...] = lu_new

    xp_buf[nxt] = jnp.dot(x_nxt_ref[0], wx_ref[...])
    c_buf[nxt] = jnp.dot(jax.nn.sigmoid(lu_new), wum_ref[...])

    idx = jnp.clip(pos - 1, 0, T - 1)
    nodes = [out_ref[s] for s in range(T)]
    for level in range(5):
        take_hi = ((idx >> level) & 1) == 1
        nodes = [jnp.where(take_hi, nodes[2 * i + 1], nodes[2 * i])
                 for i in range(len(nodes) // 2)]
    valid = (pos >= 1) & (pos <= t)
    entry = jnp.where(valid, nodes[0], 0.0)

    h_new = jnp.dot(entry, fc2a_ref[...]) + hp[:, ES:] + fc2bias_ref[...]

    wi = xp[:, ES:] + bih_ref[...]
    wh = jnp.dot(h_new, whh_ref[...]) + bhh_ref[...]
    r = jax.nn.sigmoid(wi[:, :H] + wh[:, :H])
    z = jax.nn.sigmoid(wi[:, H:2 * H] + wh[:, H:2 * H])
    n = jnp.tanh(wi[:, 2 * H:] + r * wh[:, 2 * H:])
    h_out = (1.0 - z) * n + z * h_new

    h_scr[...] = h_out
    out_ref[t] = h_out


@functools.partial(jax.jit, static_argnames=())
def kernel(input_, gumbel_u, weight_ih, weight_hh, bias, weight_im,
           weight_hm, weight_um, fc1_w, fc1_b, fc2_w, fc2_b):
    bias_ih = bias[: 3 * H].reshape(1, 3 * H)
    bias_hh = bias[3 * H:].reshape(1, 3 * H)
    fc1b = fc1_b.reshape(1, MC)
    fc2bias = fc2_b.reshape(1, H)
    fc2a = fc2_w[:ES]
    w_x = jnp.concatenate([weight_im, weight_ih], axis=1)
    w_h = jnp.concatenate([weight_hm, fc2_w[ES:]], axis=1)

    full = lambda shape: pl.BlockSpec(shape, lambda t: (0,) * len(shape))
    return pl.pallas_call(
        _step_kernel,
        grid=(T,),
        in_specs=[
            pl.BlockSpec((1, B, D), lambda t: (t, 0, 0)),
            pl.BlockSpec((1, B, D),
                         lambda t: (jnp.minimum(t + 1, T - 1), 0, 0)),
            pl.BlockSpec((1, B, MC), lambda t: (t, 0, 0)),
            full((D, ES + 3 * H)),
            full((H, 3 * H)),
            full((1, 3 * H)),
            full((1, 3 * H)),
            full((H, ES + H)),
            full((MC, ES)),
            full((ES, MC)),
            full((1, MC)),
            full((ES, H)),
            full((1, H)),
        ],
        out_specs=pl.BlockSpec((T, B, H), lambda t: (0, 0, 0)),
        out_shape=jax.ShapeDtypeStruct((T, B, H), jnp.float32),
        scratch_shapes=[
            pltpu.VMEM((B, H), jnp.float32),
            pltpu.VMEM((B, MC), jnp.float32),
            pltpu.VMEM((2, B, ES + 3 * H), jnp.float32),
            pltpu.VMEM((2, B, ES), jnp.float32),
        ],
        compiler_params=pltpu.CompilerParams(
            dimension_semantics=("arbitrary",),
        ),
    )(input_, input_, gumbel_u, w_x, weight_hh, bias_ih, bias_hh,
      w_h, weight_um, fc1_w, fc1b, fc2a, fc2bias)


# speedup vs baseline: 1.1267x; 1.0396x over previous
import functools

import jax
import jax.numpy as jnp
from jax.experimental import pallas as pl
from jax.experimental.pallas import tpu as pltpu

T, B, D, H, MC, ES = 32, 64, 256, 256, 1024, 256


def _step_kernel(x_ref, g_ref, wih_ref, whh_ref, bih_ref, bhh_ref,
                 wim_ref, whm_ref, wum_ref, fc1w_ref, fc1b_ref,
                 fc2a_ref, fc2b_ref, fc2bias_ref,
                 out_ref, h_scr, lu_scr, c_buf):
    t = pl.program_id(0)
    par = jax.lax.rem(t, 2)
    nxt = jax.lax.rem(t + 1, 2)

    @pl.when(t == 0)
    def _init():
        h_scr[...] = jnp.zeros((B, H), jnp.float32)
        lu_scr[...] = jnp.full((B, MC), -99999.0, jnp.float32)
        c_buf[0] = jnp.zeros((B, ES), jnp.float32)

    x = x_ref[0]
    h = h_scr[...]
    lu = lu_scr[...]
    c = c_buf[par]

    pre = jnp.tanh(jnp.dot(x, wim_ref[...])
                   + jnp.dot(h, whm_ref[...])
                   + c)
    read_head = jnp.dot(pre, fc1w_ref[...]) + fc1b_ref[...]
    u = g_ref[0]
    g = -jnp.log(1e-20 - jnp.log(1e-20 + u))
    logits = read_head + g

    m = jnp.max(logits, axis=1, keepdims=True)
    col = jax.lax.broadcasted_iota(jnp.int32, (B, MC), 1)
    pos = jnp.min(jnp.where(logits == m, col, MC), axis=1,
                  keepdims=True)

    lu_new = jnp.where(col == pos, -1.0, lu - 1.0)
    lu_scr[...] = lu_new
    c_buf[nxt] = jnp.dot(jax.nn.sigmoid(lu_new), wum_ref[...])

    idx = jnp.clip(pos - 1, 0, T - 1)
    nodes = [out_ref[s] for s in range(T)]
    for level in range(5):
        take_hi = ((idx >> level) & 1) == 1
        nodes = [jnp.where(take_hi, nodes[2 * i + 1], nodes[2 * i])
                 for i in range(len(nodes) // 2)]
    valid = (pos >= 1) & (pos <= t)
    entry = jnp.where(valid, nodes[0], 0.0)

    h_new = (jnp.dot(entry, fc2a_ref[...]) + jnp.dot(h, fc2b_ref[...])
             + fc2bias_ref[...])

    wi = jnp.dot(x, wih_ref[...]) + bih_ref[...]
    wh = jnp.dot(h_new, whh_ref[...]) + bhh_ref[...]
    r = jax.nn.sigmoid(wi[:, :H] + wh[:, :H])
    z = jax.nn.sigmoid(wi[:, H:2 * H] + wh[:, H:2 * H])
    n = jnp.tanh(wi[:, 2 * H:] + r * wh[:, 2 * H:])
    h_out = (1.0 - z) * n + z * h_new

    h_scr[...] = h_out
    out_ref[t] = h_out


@functools.partial(jax.jit, static_argnames=())
def kernel(input_, gumbel_u, weight_ih, weight_hh, bias, weight_im,
           weight_hm, weight_um, fc1_w, fc1_b, fc2_w, fc2_b):
    bias_ih = bias[: 3 * H].reshape(1, 3 * H)
    bias_hh = bias[3 * H:].reshape(1, 3 * H)
    fc1b = fc1_b.reshape(1, MC)
    fc2bias = fc2_b.reshape(1, H)
    fc2a = fc2_w[:ES]
    fc2b = fc2_w[ES:]

    full = lambda shape: pl.BlockSpec(shape, lambda t: (0,) * len(shape))
    return pl.pallas_call(
        _step_kernel,
        grid=(T,),
        in_specs=[
            pl.BlockSpec((1, B, D), lambda t: (t, 0, 0)),
            pl.BlockSpec((1, B, MC), lambda t: (t, 0, 0)),
            full((D, 3 * H)),
            full((H, 3 * H)),
            full((1, 3 * H)),
            full((1, 3 * H)),
            full((D, ES)),
            full((H, ES)),
            full((MC, ES)),
            full((ES, MC)),
            full((1, MC)),
            full((ES, H)),
            full((H, H)),
            full((1, H)),
        ],
        out_specs=pl.BlockSpec((T, B, H), lambda t: (0, 0, 0)),
        out_shape=jax.ShapeDtypeStruct((T, B, H), jnp.float32),
        scratch_shapes=[
            pltpu.VMEM((B, H), jnp.float32),
            pltpu.VMEM((B, MC), jnp.float32),
            pltpu.VMEM((2, B, ES), jnp.float32),
        ],
        compiler_params=pltpu.CompilerParams(
            dimension_semantics=("arbitrary",),
        ),
    )(input_, gumbel_u, weight_ih, weight_hh, bias_ih, bias_hh,
      weight_im, weight_hm, weight_um, fc1_w, fc1b, fc2a, fc2b, fc2bias)
